# baseline (speedup 1.0000x reference)
import jax
import jax.numpy as jnp
from jax.experimental import pallas as pl
from jax.experimental.pallas import tpu as pltpu

_TN = 1024


def _chamfer_body(x1_ref, x2t_ref, d1_ref, d2_ref):
    i = pl.program_id(1)
    x1 = x1_ref[0]
    x2t = x2t_ref[0]
    sq1 = jnp.sum(x1 * x1, axis=1, keepdims=True)
    sq2 = 0.25 * jnp.sum(x2t * x2t, axis=0, keepdims=True)
    g = jax.lax.dot_general(
        x1, x2t, (((1,), (0,)), ((), ())),
        preferred_element_type=jnp.float32)
    acc = (sq1 + sq2) + g
    d1_ref[0, 0, :] = jnp.maximum(jnp.min(acc, axis=1), 0.0)
    part2 = jnp.maximum(jnp.min(acc, axis=0), 0.0)

    @pl.when(i == 0)
    def _init():
        d2_ref[0, 0, :] = part2

    @pl.when(i != 0)
    def _accum():
        d2_ref[0, 0, :] = jnp.minimum(d2_ref[0, 0, :], part2)


def kernel(xyz1, xyz2):
    B, N, D = xyz1.shape
    M = xyz2.shape[1]
    xyz2t = -2.0 * jnp.swapaxes(xyz2, 1, 2)
    d1, d2 = pl.pallas_call(
        _chamfer_body,
        grid=(B, N // _TN),
        in_specs=[
            pl.BlockSpec((1, _TN, D), lambda b, i: (b, i, 0)),
            pl.BlockSpec((1, D, M), lambda b, i: (b, 0, 0)),
        ],
        out_specs=[
            pl.BlockSpec((1, 1, _TN), lambda b, i: (b, 0, i)),
            pl.BlockSpec((1, 1, M), lambda b, i: (b, 0, 0)),
        ],
        out_shape=[
            jax.ShapeDtypeStruct((B, 1, N), jnp.float32),
            jax.ShapeDtypeStruct((B, 1, M), jnp.float32),
        ],
        compiler_params=pltpu.CompilerParams(
            dimension_semantics=("parallel", "arbitrary")),
    )(xyz1, xyz2t)
    return jnp.mean(d1) + jnp.mean(d2)


# speedup vs baseline: 1.2093x; 1.2093x over previous
import jax
import jax.numpy as jnp
from jax.experimental import pallas as pl
from jax.experimental.pallas import tpu as pltpu

_TN = 512


def _chamfer_body(x1_ref, x2t_ref, d1_ref, d2_ref):
    i = pl.program_id(1)
    x1 = x1_ref[0]
    x2t = x2t_ref[0]
    sq1 = jnp.sum(x1 * x1, axis=1, keepdims=True)
    sq2 = 0.25 * jnp.sum(x2t * x2t, axis=0, keepdims=True)
    g = jax.lax.dot_general(
        x1, x2t, (((1,), (0,)), ((), ())),
        preferred_element_type=jnp.float32)
    acc = (sq1 + sq2) + g
    p = acc
    while p.shape[1] > 128:
        h = p.shape[1] // 2
        p = jnp.minimum(p[:, :h], p[:, h:])
    d1_ref[0, 0, :] = jnp.maximum(jnp.min(p.T, axis=0), 0.0)
    part2 = jnp.maximum(jnp.min(acc, axis=0), 0.0)

    @pl.when(i == 0)
    def _init():
        d2_ref[0, 0, :] = part2

    @pl.when(i != 0)
    def _accum():
        d2_ref[0, 0, :] = jnp.minimum(d2_ref[0, 0, :], part2)


def kernel(xyz1, xyz2):
    B, N, D = xyz1.shape
    M = xyz2.shape[1]
    xyz2t = -2.0 * jnp.swapaxes(xyz2, 1, 2)
    d1, d2 = pl.pallas_call(
        _chamfer_body,
        grid=(B, N // _TN),
        in_specs=[
            pl.BlockSpec((1, _TN, D), lambda b, i: (b, i, 0)),
            pl.BlockSpec((1, D, M), lambda b, i: (b, 0, 0)),
        ],
        out_specs=[
            pl.BlockSpec((1, 1, _TN), lambda b, i: (b, 0, i)),
            pl.BlockSpec((1, 1, M), lambda b, i: (b, 0, 0)),
        ],
        out_shape=[
            jax.ShapeDtypeStruct((B, 1, N), jnp.float32),
            jax.ShapeDtypeStruct((B, 1, M), jnp.float32),
        ],
        compiler_params=pltpu.CompilerParams(
            dimension_semantics=("parallel", "arbitrary")),
    )(xyz1, xyz2t)
    return jnp.mean(d1) + jnp.mean(d2)


# speedup vs baseline: 1.2457x; 1.0301x over previous
import functools

import jax
import jax.numpy as jnp
from jax.experimental import pallas as pl
from jax.experimental.pallas import tpu as pltpu

_TN = 512


def _chamfer_body(x1_ref, x2t_ref, out_ref, d2_scr, *, inv1, inv2):
    i = pl.program_id(1)
    ni = pl.num_programs(1)
    x1 = x1_ref[0]
    x2t = x2t_ref[0]
    sq1 = jnp.sum(x1 * x1, axis=1, keepdims=True)
    sq2 = 0.25 * jnp.sum(x2t * x2t, axis=0, keepdims=True)
    g = jax.lax.dot_general(
        x1, x2t, (((1,), (0,)), ((), ())),
        preferred_element_type=jnp.float32)
    acc = (sq1 + sq2) + g
    p = acc
    while p.shape[1] > 128:
        h = p.shape[1] // 2
        p = jnp.minimum(p[:, :h], p[:, h:])
    d1_tile = jnp.maximum(jnp.min(p.T, axis=0, keepdims=True), 0.0)
    s1 = jnp.sum(d1_tile, axis=1, keepdims=True) * inv1
    part2 = jnp.min(acc, axis=0, keepdims=True)

    @pl.when(i == 0)
    def _init():
        d2_scr[:, :] = part2
        out_ref[0] = s1

    @pl.when(i != 0)
    def _accum():
        d2_scr[:, :] = jnp.minimum(d2_scr[:, :], part2)
        out_ref[0] = out_ref[0] + s1

    @pl.when(i == ni - 1)
    def _final():
        d2f = jnp.maximum(d2_scr[:, :], 0.0)
        out_ref[0] = out_ref[0] + jnp.sum(d2f, axis=1, keepdims=True) * inv2


def kernel(xyz1, xyz2):
    B, N, D = xyz1.shape
    M = xyz2.shape[1]
    xyz2t = -2.0 * jnp.swapaxes(xyz2, 1, 2)
    out = pl.pallas_call(
        functools.partial(_chamfer_body, inv1=1.0 / (B * N), inv2=1.0 / (B * M)),
        grid=(B, N // _TN),
        in_specs=[
            pl.BlockSpec((1, _TN, D), lambda b, i: (b, i, 0)),
            pl.BlockSpec((1, D, M), lambda b, i: (b, 0, 0)),
        ],
        out_specs=pl.BlockSpec((1, 1, 1), lambda b, i: (b, 0, 0)),
        out_shape=jax.ShapeDtypeStruct((B, 1, 1), jnp.float32),
        scratch_shapes=[pltpu.VMEM((1, M), jnp.float32)],
        compiler_params=pltpu.CompilerParams(
            dimension_semantics=("parallel", "arbitrary")),
    )(xyz1, xyz2t)
    return jnp.sum(out)


# speedup vs baseline: 1.2808x; 1.0282x over previous
import functools

import jax
import jax.numpy as jnp
from jax.experimental import pallas as pl
from jax.experimental.pallas import tpu as pltpu

_TN = 512


def _chamfer_body(x1_ref, x2t_ref, out_ref, d2_scr, s1_scr, *, inv1, inv2):
    i = pl.program_id(1)
    ni = pl.num_programs(1)
    x1 = x1_ref[0]
    x2t = x2t_ref[0]
    sq1 = jnp.sum(x1 * x1, axis=1, keepdims=True)
    sq2 = 0.25 * jnp.sum(x2t * x2t, axis=0, keepdims=True)
    g = jax.lax.dot_general(
        x1, x2t, (((1,), (0,)), ((), ())),
        preferred_element_type=jnp.float32)
    acc = (sq1 + sq2) + g
    p = acc
    while p.shape[1] > 128:
        h = p.shape[1] // 2
        p = jnp.minimum(p[:, :h], p[:, h:])
    d1_tile = jnp.maximum(jnp.min(p.T, axis=0, keepdims=True), 0.0)
    f = d1_tile
    while f.shape[1] > 128:
        h = f.shape[1] // 2
        f = f[:, :h] + f[:, h:]
    part2 = jnp.min(acc, axis=0, keepdims=True)

    @pl.when(i == 0)
    def _init():
        d2_scr[:, :] = part2
        s1_scr[:, :] = f

    @pl.when(i != 0)
    def _accum():
        d2_scr[:, :] = jnp.minimum(d2_scr[:, :], part2)
        s1_scr[:, :] = s1_scr[:, :] + f

    @pl.when(i == ni - 1)
    def _final():
        s1 = jnp.sum(s1_scr[:, :], axis=1, keepdims=True) * inv1
        d2f = jnp.maximum(d2_scr[:, :], 0.0)
        out_ref[0] = s1 + jnp.sum(d2f, axis=1, keepdims=True) * inv2


def kernel(xyz1, xyz2):
    B, N, D = xyz1.shape
    M = xyz2.shape[1]
    xyz2t = -2.0 * jnp.swapaxes(xyz2, 1, 2)
    out = pl.pallas_call(
        functools.partial(_chamfer_body, inv1=1.0 / (B * N), inv2=1.0 / (B * M)),
        grid=(B, N // _TN),
        in_specs=[
            pl.BlockSpec((1, _TN, D), lambda b, i: (b, i, 0)),
            pl.BlockSpec((1, D, M), lambda b, i: (b, 0, 0)),
        ],
        out_specs=pl.BlockSpec((1, 1, 1), lambda b, i: (b, 0, 0)),
        out_shape=jax.ShapeDtypeStruct((B, 1, 1), jnp.float32),
        scratch_shapes=[pltpu.VMEM((1, M), jnp.float32),
                        pltpu.VMEM((1, 128), jnp.float32)],
        compiler_params=pltpu.CompilerParams(
            dimension_semantics=("parallel", "arbitrary")),
    )(xyz1, xyz2t)
    return jnp.sum(out)


# speedup vs baseline: 1.3892x; 1.0846x over previous
import functools

import jax
import jax.numpy as jnp
from jax.experimental import pallas as pl
from jax.experimental.pallas import tpu as pltpu

_TN = 1024


def _chamfer_body(x1_ref, x2t_ref, out_ref, d2_scr, s1_scr, *, inv1, inv2):
    i = pl.program_id(1)
    ni = pl.num_programs(1)
    x1 = x1_ref[0]
    x2t = x2t_ref[0]
    sq1 = jnp.sum(x1 * x1, axis=1, keepdims=True)
    sq2 = 0.25 * jnp.sum(x2t * x2t, axis=0, keepdims=True)
    g = jax.lax.dot_general(
        x1, x2t, (((1,), (0,)), ((), ())),
        preferred_element_type=jnp.float32)
    acc = (sq1 + sq2) + g
    p = acc
    while p.shape[1] > 128:
        h = p.shape[1] // 2
        p = jnp.minimum(p[:, :h], p[:, h:])
    d1_tile = jnp.maximum(jnp.min(p.T, axis=0, keepdims=True), 0.0)
    f = d1_tile
    while f.shape[1] > 128:
        h = f.shape[1] // 2
        f = f[:, :h] + f[:, h:]
    part2 = jnp.min(acc, axis=0, keepdims=True)

    @pl.when(i == 0)
    def _init():
        d2_scr[:, :] = part2
        s1_scr[:, :] = f

    @pl.when(i != 0)
    def _accum():
        d2_scr[:, :] = jnp.minimum(d2_scr[:, :], part2)
        s1_scr[:, :] = s1_scr[:, :] + f

    @pl.when(i == ni - 1)
    def _final():
        s1 = jnp.sum(s1_scr[:, :], axis=1, keepdims=True) * inv1
        d2f = jnp.maximum(d2_scr[:, :], 0.0)
        out_ref[0] = s1 + jnp.sum(d2f, axis=1, keepdims=True) * inv2


def kernel(xyz1, xyz2):
    B, N, D = xyz1.shape
    M = xyz2.shape[1]
    xyz2t = -2.0 * jnp.swapaxes(xyz2, 1, 2)
    out = pl.pallas_call(
        functools.partial(_chamfer_body, inv1=1.0 / (B * N), inv2=1.0 / (B * M)),
        grid=(B, N // _TN),
        in_specs=[
            pl.BlockSpec((1, _TN, D), lambda b, i: (b, i, 0)),
            pl.BlockSpec((1, D, M), lambda b, i: (b, 0, 0)),
        ],
        out_specs=pl.BlockSpec((1, 1, 1), lambda b, i: (b, 0, 0)),
        out_shape=jax.ShapeDtypeStruct((B, 1, 1), jnp.float32),
        scratch_shapes=[pltpu.VMEM((1, M), jnp.float32),
                        pltpu.VMEM((1, 128), jnp.float32)],
        compiler_params=pltpu.CompilerParams(
            dimension_semantics=("parallel", "arbitrary")),
    )(xyz1, xyz2t)
    return jnp.sum(out)


# speedup vs baseline: 1.4523x; 1.0454x over previous
import functools

import jax
import jax.numpy as jnp
from jax.experimental import pallas as pl
from jax.experimental.pallas import tpu as pltpu

_TN = 2048


def _chamfer_body(x1_ref, x2t_ref, out_ref, d2_scr, s1_scr, *, inv1, inv2):
    i = pl.program_id(1)
    ni = pl.num_programs(1)
    x1 = x1_ref[0]
    x2t = x2t_ref[0]
    sq1 = jnp.sum(x1 * x1, axis=1, keepdims=True)
    sq2 = 0.25 * jnp.sum(x2t * x2t, axis=0, keepdims=True)
    g = jax.lax.dot_general(
        x1, x2t, (((1,), (0,)), ((), ())),
        preferred_element_type=jnp.float32)
    acc = (sq1 + sq2) + g
    p = acc
    while p.shape[1] > 128:
        h = p.shape[1] // 2
        p = jnp.minimum(p[:, :h], p[:, h:])
    d1_tile = jnp.maximum(jnp.min(p.T, axis=0, keepdims=True), 0.0)
    f = d1_tile
    while f.shape[1] > 128:
        h = f.shape[1] // 2
        f = f[:, :h] + f[:, h:]
    part2 = jnp.min(acc, axis=0, keepdims=True)

    @pl.when(i == 0)
    def _init():
        d2_scr[:, :] = part2
        s1_scr[:, :] = f

    @pl.when(i != 0)
    def _accum():
        d2_scr[:, :] = jnp.minimum(d2_scr[:, :], part2)
        s1_scr[:, :] = s1_scr[:, :] + f

    @pl.when(i == ni - 1)
    def _final():
        s1 = jnp.sum(s1_scr[:, :], axis=1, keepdims=True) * inv1
        d2f = jnp.maximum(d2_scr[:, :], 0.0)
        out_ref[0] = s1 + jnp.sum(d2f, axis=1, keepdims=True) * inv2


def kernel(xyz1, xyz2):
    B, N, D = xyz1.shape
    M = xyz2.shape[1]
    xyz2t = -2.0 * jnp.swapaxes(xyz2, 1, 2)
    out = pl.pallas_call(
        functools.partial(_chamfer_body, inv1=1.0 / (B * N), inv2=1.0 / (B * M)),
        grid=(B, N // _TN),
        in_specs=[
            pl.BlockSpec((1, _TN, D), lambda b, i: (b, i, 0)),
            pl.BlockSpec((1, D, M), lambda b, i: (b, 0, 0)),
        ],
        out_specs=pl.BlockSpec((1, 1, 1), lambda b, i: (b, 0, 0)),
        out_shape=jax.ShapeDtypeStruct((B, 1, 1), jnp.float32),
        scratch_shapes=[pltpu.VMEM((1, M), jnp.float32),
                        pltpu.VMEM((1, 128), jnp.float32)],
        compiler_params=pltpu.CompilerParams(
            dimension_semantics=("parallel", "arbitrary")),
    )(xyz1, xyz2t)
    return jnp.sum(out)
